# baseline (speedup 1.0000x reference)
import functools

import jax
import jax.numpy as jnp
from jax import lax
from jax.experimental import pallas as pl
from jax.experimental.pallas import tpu as pltpu
from jax.experimental.pallas import tpu_sc as plsc

VOCAB = 1000000
EMBED_DIM = 64
BATCH = 4096
_L = 16


def _sc_gather_pair(target, context, in_tab, out_tab):
    info = plsc.get_sparse_core_info()
    nw = info.num_cores * info.num_subcores
    bw = BATCH // nw
    mesh = plsc.VectorSubcoreMesh(core_axis_name="c", subcore_axis_name="s")

    @functools.partial(
        pl.kernel,
        out_type=(
            jax.ShapeDtypeStruct((BATCH, EMBED_DIM), jnp.float32),
            jax.ShapeDtypeStruct((BATCH, EMBED_DIM), jnp.float32),
        ),
        mesh=mesh,
        compiler_params=pltpu.CompilerParams(needs_layout_passes=False),
        scratch_types=[
            pltpu.VMEM((bw,), jnp.int32),
            pltpu.SemaphoreType.DMA,
        ],
    )
    def gather_kernel(tgt_hbm, ctx_hbm, in_tab_hbm, out_tab_hbm, tgt_rows_hbm,
                      ctx_rows_hbm, idx_v, sem):
        wid = lax.axis_index("s") * info.num_cores + lax.axis_index("c")
        base = wid * bw
        iota = lax.iota(jnp.int32, _L)

        def one_table(idx_hbm, tab, rows_out_hbm):
            pltpu.sync_copy(idx_hbm.at[pl.ds(base, bw)], idx_v)

            def group_body(g, carry):
                idxg = idx_v[pl.ds(g * _L, _L)]
                for l in range(_L):
                    ri = jnp.sum(jnp.where(iota == l, idxg, 0))
                    pltpu.async_copy(
                        tab.at[pl.ds(ri, 1)],
                        rows_out_hbm.at[pl.ds(base + g * _L + l, 1)], sem)
                return carry

            lax.fori_loop(0, bw // _L, group_body, 0)
            pltpu.make_async_copy(
                tab.at[pl.ds(0, bw)],
                rows_out_hbm.at[pl.ds(base, bw)], sem).wait()

        one_table(tgt_hbm, in_tab_hbm, tgt_rows_hbm)
        one_table(ctx_hbm, out_tab_hbm, ctx_rows_hbm)

    return gather_kernel(target, context, in_tab, out_tab)


def _scores_matmul(tgt16, ctx16_t):
    bm = 256

    def mm(a_ref, b_ref, o_ref):
        o_ref[...] = lax.dot_general(
            a_ref[...], b_ref[...],
            dimension_numbers=(((1,), (0,)), ((), ())),
            preferred_element_type=jnp.float32,
        )

    return pl.pallas_call(
        mm,
        grid=(BATCH // bm,),
        in_specs=[
            pl.BlockSpec((bm, EMBED_DIM), lambda i: (i, 0)),
            pl.BlockSpec((EMBED_DIM, BATCH), lambda i: (0, 0)),
        ],
        out_specs=pl.BlockSpec((bm, BATCH), lambda i: (i, 0)),
        out_shape=jax.ShapeDtypeStruct((BATCH, BATCH), jnp.float32),
    )(tgt16, ctx16_t)


def kernel(target, context, in_embed_weight, out_embed_weight):
    tgt_rows, ctx_rows = _sc_gather_pair(
        target, context, in_embed_weight, out_embed_weight)
    tgt16 = tgt_rows.astype(jnp.bfloat16)
    ctx16_t = ctx_rows.astype(jnp.bfloat16).T
    return _scores_matmul(tgt16, ctx16_t)


# speedup vs baseline: 1.0017x; 1.0017x over previous
import functools

import jax
import jax.numpy as jnp
from jax import lax
from jax.experimental import pallas as pl
from jax.experimental.pallas import tpu as pltpu
from jax.experimental.pallas import tpu_sc as plsc

VOCAB = 1000000
EMBED_DIM = 64
BATCH = 4096
_L = 16


def _sc_gather_pair(target, context, in_tab, out_tab):
    info = plsc.get_sparse_core_info()
    nw = info.num_cores * info.num_subcores
    bw = BATCH // nw
    mesh = plsc.VectorSubcoreMesh(core_axis_name="c", subcore_axis_name="s")

    @functools.partial(
        pl.kernel,
        out_type=(
            jax.ShapeDtypeStruct((BATCH, EMBED_DIM), jnp.float32),
            jax.ShapeDtypeStruct((BATCH, EMBED_DIM), jnp.float32),
        ),
        mesh=mesh,
        compiler_params=pltpu.CompilerParams(
            needs_layout_passes=False, use_tc_tiling_on_sc=True),
        scratch_types=[
            pltpu.VMEM((bw,), jnp.int32),
            pltpu.SemaphoreType.DMA,
        ],
    )
    def gather_kernel(tgt_hbm, ctx_hbm, in_tab_hbm, out_tab_hbm, tgt_rows_hbm,
                      ctx_rows_hbm, idx_v, sem):
        wid = lax.axis_index("s") * info.num_cores + lax.axis_index("c")
        base = wid * bw
        iota = lax.iota(jnp.int32, _L)

        def one_table(idx_hbm, tab, rows_out_hbm):
            pltpu.sync_copy(idx_hbm.at[pl.ds(base, bw)], idx_v)

            def group_body(g, carry):
                idxg = idx_v[pl.ds(g * _L, _L)]
                for l in range(_L):
                    ri = jnp.sum(jnp.where(iota == l, idxg, 0))
                    pltpu.async_copy(
                        tab.at[pl.ds(ri, 1)],
                        rows_out_hbm.at[pl.ds(base + g * _L + l, 1)], sem)
                return carry

            lax.fori_loop(0, bw // _L, group_body, 0)
            pltpu.make_async_copy(
                tab.at[pl.ds(0, bw)],
                rows_out_hbm.at[pl.ds(base, bw)], sem).wait()

        one_table(tgt_hbm, in_tab_hbm, tgt_rows_hbm)
        one_table(ctx_hbm, out_tab_hbm, ctx_rows_hbm)

    return gather_kernel(target, context, in_tab, out_tab)


def _scores_matmul(tgt16, ctx16_t):
    bm = 256

    def mm(a_ref, b_ref, o_ref):
        o_ref[...] = lax.dot_general(
            a_ref[...], b_ref[...],
            dimension_numbers=(((1,), (0,)), ((), ())),
            preferred_element_type=jnp.float32,
        )

    return pl.pallas_call(
        mm,
        grid=(BATCH // bm,),
        in_specs=[
            pl.BlockSpec((bm, EMBED_DIM), lambda i: (i, 0)),
            pl.BlockSpec((EMBED_DIM, BATCH), lambda i: (0, 0)),
        ],
        out_specs=pl.BlockSpec((bm, BATCH), lambda i: (i, 0)),
        out_shape=jax.ShapeDtypeStruct((BATCH, BATCH), jnp.float32),
    )(tgt16, ctx16_t)


def kernel(target, context, in_embed_weight, out_embed_weight):
    tgt_rows, ctx_rows = _sc_gather_pair(
        target, context, in_embed_weight, out_embed_weight)
    tgt16 = tgt_rows.astype(jnp.bfloat16)
    ctx16_t = ctx_rows.astype(jnp.bfloat16).T
    return _scores_matmul(tgt16, ctx16_t)


# speedup vs baseline: 1.1287x; 1.1268x over previous
import functools

import jax
import jax.numpy as jnp
from jax import lax
from jax.experimental import pallas as pl
from jax.experimental.pallas import tpu as pltpu

VOCAB = 1000000
EMBED_DIM = 64
BATCH = 4096


def _gather_rows(target, context, in_tab, out_tab):
    def body(tgt_s, ctx_s, in_hbm, out_hbm, tgt_rows, ctx_rows, sem_t, sem_c):
        def one_table(idx_s, tab, rows, sem):
            def row_body(r, carry):
                ri = idx_s[r]
                pltpu.make_async_copy(
                    tab.at[pl.ds(ri, 1)], rows.at[pl.ds(r, 1)], sem).start()
                return carry

            lax.fori_loop(0, BATCH, row_body, 0)

        one_table(tgt_s, in_hbm, tgt_rows, sem_t)
        one_table(ctx_s, out_hbm, ctx_rows, sem_c)
        pltpu.make_async_copy(
            in_hbm.at[pl.ds(0, BATCH)], tgt_rows, sem_t).wait()
        pltpu.make_async_copy(
            out_hbm.at[pl.ds(0, BATCH)], ctx_rows, sem_c).wait()

    return pl.pallas_call(
        body,
        in_specs=[
            pl.BlockSpec(memory_space=pltpu.SMEM),
            pl.BlockSpec(memory_space=pltpu.SMEM),
            pl.BlockSpec(memory_space=pl.ANY),
            pl.BlockSpec(memory_space=pl.ANY),
        ],
        out_specs=[
            pl.BlockSpec(memory_space=pltpu.VMEM),
            pl.BlockSpec(memory_space=pltpu.VMEM),
        ],
        out_shape=[
            jax.ShapeDtypeStruct((BATCH, EMBED_DIM), jnp.float32),
            jax.ShapeDtypeStruct((BATCH, EMBED_DIM), jnp.float32),
        ],
        scratch_shapes=[pltpu.SemaphoreType.DMA, pltpu.SemaphoreType.DMA],
    )(target, context, in_tab, out_tab)


def _scores_matmul(tgt_rows, ctx_rows):
    bm = 256

    def mm(a_ref, b_ref, o_ref):
        a16 = a_ref[...].astype(jnp.bfloat16)
        b16 = b_ref[...].astype(jnp.bfloat16)
        o_ref[...] = lax.dot_general(
            a16, b16,
            dimension_numbers=(((1,), (1,)), ((), ())),
            preferred_element_type=jnp.float32,
        )

    return pl.pallas_call(
        mm,
        grid=(BATCH // bm,),
        in_specs=[
            pl.BlockSpec((bm, EMBED_DIM), lambda i: (i, 0)),
            pl.BlockSpec((BATCH, EMBED_DIM), lambda i: (0, 0)),
        ],
        out_specs=pl.BlockSpec((bm, BATCH), lambda i: (i, 0)),
        out_shape=jax.ShapeDtypeStruct((BATCH, BATCH), jnp.float32),
    )(tgt_rows, ctx_rows)


def kernel(target, context, in_embed_weight, out_embed_weight):
    tgt_rows, ctx_rows = _gather_rows(
        target, context, in_embed_weight, out_embed_weight)
    return _scores_matmul(tgt_rows, ctx_rows)
